# baseline (speedup 1.0000x reference)
import functools

import jax
import jax.numpy as jnp
from jax import lax
from jax.experimental import pallas as pl
from jax.experimental.pallas import tpu as pltpu

_CODE_DIM = 64
_NUM_CODES = 1024
_COMMIT = 0.25
_BB = 8
_HW = 576
_NBLK = _BB * _HW


def _vq_body(z_ref, cb_ref, cbm2_ref, idx_ref, zq_ref, loss_ref, perp_ref,
             acc_ref, counts_ref, *, num_steps, n_total):
    i = pl.program_id(0)
    zb = jnp.concatenate([z_ref[s] for s in range(_BB)], axis=1)
    cb = cb_ref[...]
    cbm2 = cbm2_ref[...]

    csum = jnp.sum(cb * cb, axis=1, keepdims=True)
    zsq = jnp.sum(zb * zb, axis=0, keepdims=True)

    mm2 = lax.dot_general(cbm2, zb, (((1,), (0,)), ((), ())),
                          preferred_element_type=jnp.float32)
    distT = (zsq + csum) + mm2

    minv = jnp.min(distT, axis=0, keepdims=True)
    kio = lax.broadcasted_iota(jnp.int32, (_NUM_CODES, _NBLK), 0)
    hit = distT == minv
    idx = jnp.min(jnp.where(hit, kio, _NUM_CODES), axis=0,
                  keepdims=True)
    for s in range(_BB):
        idx_ref[s] = idx[:, s * _HW:(s + 1) * _HW].astype(jnp.int32)

    oh = jnp.where(kio == idx, 1.0, 0.0).astype(jnp.float32)
    zq = lax.dot_general(cb, oh, (((0,), (0,)), ((), ())),
                         preferred_element_type=jnp.float32)
    for s in range(_BB):
        zq_ref[s] = zq[:, s * _HW:(s + 1) * _HW]

    @pl.when(i == 0)
    def _init():
        acc_ref[0, 0] = 0.0
        counts_ref[...] = jnp.zeros_like(counts_ref)

    acc_ref[0, 0] += jnp.sum(minv)
    counts_ref[...] += jnp.sum(oh, axis=1, keepdims=True)

    @pl.when(i == num_steps - 1)
    def _finish():
        loss_ref[0, 0] = (1.0 + _COMMIT) * acc_ref[0, 0] / float(n_total * _CODE_DIM)
        avg = counts_ref[...] * (1.0 / float(n_total))
        ent = -jnp.sum(avg * jnp.log(avg + 1e-10))
        perp_ref[0, 0] = jnp.exp(ent)


def kernel(z, codebook):
    B, C, H, W = z.shape
    hw = H * W
    n_total = B * hw
    num_steps = B // _BB
    z3 = z.reshape(B, C, hw)
    cbm2 = -2.0 * codebook

    idx3, zq3, loss2, perp2 = pl.pallas_call(
        functools.partial(_vq_body, num_steps=num_steps, n_total=n_total),
        grid=(num_steps,),
        in_specs=[
            pl.BlockSpec((_BB, C, hw), lambda i: (i, 0, 0)),
            pl.BlockSpec((_NUM_CODES, _CODE_DIM), lambda i: (0, 0)),
            pl.BlockSpec((_NUM_CODES, _CODE_DIM), lambda i: (0, 0)),
        ],
        out_specs=[
            pl.BlockSpec((_BB, 1, hw), lambda i: (i, 0, 0)),
            pl.BlockSpec((_BB, C, hw), lambda i: (i, 0, 0)),
            pl.BlockSpec(memory_space=pltpu.SMEM, block_shape=(1, 1),
                         index_map=lambda i: (0, 0)),
            pl.BlockSpec(memory_space=pltpu.SMEM, block_shape=(1, 1),
                         index_map=lambda i: (0, 0)),
        ],
        out_shape=[
            jax.ShapeDtypeStruct((B, 1, hw), jnp.int32),
            jax.ShapeDtypeStruct((B, C, hw), jnp.float32),
            jax.ShapeDtypeStruct((1, 1), jnp.float32),
            jax.ShapeDtypeStruct((1, 1), jnp.float32),
        ],
        scratch_shapes=[
            pltpu.SMEM((1, 1), jnp.float32),
            pltpu.VMEM((_NUM_CODES, 1), jnp.float32),
        ],
    )(z3, codebook, cbm2)

    z_q = zq3.reshape(B, C, H, W)
    indices = idx3.reshape(B, H, W)
    return (z_q, loss2[0, 0], (indices, perp2[0, 0]))
